# baseline (speedup 1.0000x reference)
import functools

import jax
import jax.numpy as jnp
from jax import lax
from jax.experimental import pallas as pl
from jax.experimental.pallas import tpu as pltpu
from jax.experimental.pallas import tpu_sc as plsc

B, C, N_IN, N_OUT, K = 2, 256, 50000, 12500, 8
NW = 32
CP = 4
ROWS = CP * K
SLAB = B * C
BLK = 128
MB = 3
MAINC = MB * BLK // CP
TAILBASE = NW * MB * BLK
TAILCH = (N_OUT - TAILBASE) // CP
PPT = MB * BLK + 2 * CP


def _body(tab_hbm, idxt_hbm, tailg_hbm, out_hbm, idxk_v, idx_v, idx2_v,
          rows_v, stage_v, gsem, osem):
    wid = lax.axis_index("c") * 16 + lax.axis_index("s")

    for i in range(MB):
        pltpu.sync_copy(idxt_hbm.at[:, pl.ds((wid + NW * i) * BLK, BLK)],
                        idxk_v.at[:, pl.ds(i * BLK, BLK)])
    pltpu.sync_copy(tailg_hbm.at[pl.ds(wid * ROWS, ROWS)],
                    idx_v.at[pl.ds(MAINC * ROWS, ROWS)])

    @pl.when(wid + NW < TAILCH)
    def _tail2():
        pltpu.sync_copy(tailg_hbm.at[pl.ds((wid + NW) * ROWS, ROWS)],
                        idx_v.at[pl.ds((MAINC + 1) * ROWS, ROWS)])

    lanes8 = lax.iota(jnp.int32, 16) * 8

    def repack(g, _):
        for k in range(K):
            v = idxk_v[k, pl.ds(g * 16, 16)]
            pos = lanes8 + (g * 128 + k)
            plsc.store_scatter(idx_v, [pos], v)
            plsc.store_scatter(idx2_v, [pos], v + N_IN)
        return _

    lax.fori_loop(0, MB * BLK // 16, repack, 0)

    def tshift(i, _):
        idx2_v[pl.ds(MAINC * ROWS + i * 16, 16)] = (
            idx_v[pl.ds(MAINC * ROWS + i * 16, 16)] + N_IN)
        return _

    lax.fori_loop(0, 2 * ROWS // 16, tshift, 0)
    idxs = (idx_v, idx2_v)

    def out_word(c):
        main = (wid + NW * (c // (BLK // CP))) * BLK + lax.rem(c, BLK // CP) * CP
        tail = TAILBASE + (wid + NW * (c - MAINC)) * CP
        return jnp.where(c < MAINC, main, tail) * SLAB

    def gather(c, buf):
        for b in range(B):
            pltpu.async_copy(
                tab_hbm.at[idxs[b].at[pl.ds(c * ROWS, ROWS)]],
                rows_v.at[buf, pl.ds(b * ROWS, ROWS)],
                gsem.at[buf],
            )

    def gwait(c, buf):
        for b in range(B):
            pltpu.make_async_copy(
                tab_hbm.at[idxs[b].at[pl.ds(c * ROWS, ROWS)]],
                rows_v.at[buf, pl.ds(b * ROWS, ROWS)],
                gsem.at[buf],
            ).wait()

    def compute(c, buf):
        def do_v(v, _):
            coff = (v // 8) * 256 + (v % 8) * 16
            for p in range(CP):
                for b in range(B):
                    acc = None
                    for k in range(K):
                        r = rows_v[buf, b * ROWS + p * K + k, pl.ds(v * 16, 16)]
                        acc = r if acc is None else jnp.maximum(acc, r)
                    stage_v[buf, pl.ds(p * SLAB + b * 128 + coff, 16)] = acc
            return _

        lax.fori_loop(0, 16, do_v, 0, unroll=2)
        pltpu.async_copy(
            stage_v.at[buf],
            out_hbm.at[pl.ds(out_word(c), CP * SLAB)],
            osem.at[buf],
        )

    def owait(c, buf):
        pltpu.make_async_copy(
            stage_v.at[buf],
            out_hbm.at[pl.ds(out_word(c), CP * SLAB)],
            osem.at[buf],
        ).wait()

    ncr = MAINC + 1 + jnp.where(wid + NW < TAILCH, 1, 0)

    gather(0, 0)
    gather(1, 1)
    gather(2, 2)

    def step(c, _):
        buf = lax.rem(c, 4)

        @pl.when(c + 3 < ncr)
        def _prefetch():
            gather(c + 3, lax.rem(c + 3, 4))

        gwait(c, buf)

        @pl.when(c >= 4)
        def _drain_out():
            owait(c - 4, buf)

        compute(c, buf)
        return _

    lax.fori_loop(0, ncr, step, 0)
    owait(ncr - 4, lax.rem(ncr - 4, 4))
    owait(ncr - 3, lax.rem(ncr - 3, 4))
    owait(ncr - 2, lax.rem(ncr - 2, 4))
    owait(ncr - 1, lax.rem(ncr - 1, 4))


_sc_call = functools.partial(
    pl.kernel,
    out_type=jax.ShapeDtypeStruct((N_OUT * SLAB,), jnp.float32),
    mesh=plsc.VectorSubcoreMesh(core_axis_name="c", subcore_axis_name="s"),
    compiler_params=pltpu.CompilerParams(
        needs_layout_passes=False, use_tc_tiling_on_sc=True
    ),
    scratch_types=[
        pltpu.VMEM((K, MB * BLK), jnp.int32),
        pltpu.VMEM((PPT * K,), jnp.int32),
        pltpu.VMEM((PPT * K,), jnp.int32),
        pltpu.VMEM((4, B * ROWS, C), jnp.float32),
        pltpu.VMEM((4, CP * SLAB), jnp.float32),
        pltpu.SemaphoreType.DMA((4,)),
        pltpu.SemaphoreType.DMA((4,)),
    ],
)(_body)


def kernel(x, idx):
    tab = x.transpose(0, 2, 1).reshape(B * N_IN, C)
    idxt = idx.T
    tailg = idx[TAILBASE:].reshape(-1)
    out = _sc_call(tab, idxt, tailg)
    o4 = out.reshape(N_OUT, 2, B, 128)
    return o4.transpose(2, 1, 3, 0).reshape(B, C, N_OUT)


# speedup vs baseline: 1.0044x; 1.0044x over previous
import functools

import jax
import jax.numpy as jnp
from jax import lax
from jax.experimental import pallas as pl
from jax.experimental.pallas import tpu as pltpu
from jax.experimental.pallas import tpu_sc as plsc

B, C, N_IN, N_OUT, K = 2, 256, 50000, 12500, 8
NW = 32
CP = 4
ROWS = CP * K
SLAB = B * C
BLK = 128
MB = 3
MAINC = MB * BLK // CP
TAILBASE = NW * MB * BLK
TAILCH = (N_OUT - TAILBASE) // CP
PPT = MB * BLK + 2 * CP


def _body(tab_hbm, idxt_hbm, tailg_hbm, out_hbm, idxk_v, idx_v, idx2_v,
          rows_v, stage_v, gsem, osem):
    wid = lax.axis_index("c") * 16 + lax.axis_index("s")

    for i in range(MB):
        pltpu.sync_copy(idxt_hbm.at[:, pl.ds((wid + NW * i) * BLK, BLK)],
                        idxk_v.at[:, pl.ds(i * BLK, BLK)])
    pltpu.sync_copy(tailg_hbm.at[pl.ds(wid * ROWS, ROWS)],
                    idx_v.at[pl.ds(MAINC * ROWS, ROWS)])

    @pl.when(wid + NW < TAILCH)
    def _tail2():
        pltpu.sync_copy(tailg_hbm.at[pl.ds((wid + NW) * ROWS, ROWS)],
                        idx_v.at[pl.ds((MAINC + 1) * ROWS, ROWS)])

    lanes8 = lax.iota(jnp.int32, 16) * 8

    def repack(g, _):
        for k in range(K):
            v = idxk_v[k, pl.ds(g * 16, 16)]
            pos = lanes8 + (g * 128 + k)
            plsc.store_scatter(idx_v, [pos], v)
            plsc.store_scatter(idx2_v, [pos], v + N_IN)
        return _

    lax.fori_loop(0, MB * BLK // 16, repack, 0)

    def tshift(i, _):
        idx2_v[pl.ds(MAINC * ROWS + i * 16, 16)] = (
            idx_v[pl.ds(MAINC * ROWS + i * 16, 16)] + N_IN)
        return _

    lax.fori_loop(0, 2 * ROWS // 16, tshift, 0)
    idxs = (idx_v, idx2_v)

    def out_word(c):
        main = (wid + NW * (c // (BLK // CP))) * BLK + lax.rem(c, BLK // CP) * CP
        tail = TAILBASE + (wid + NW * (c - MAINC)) * CP
        return jnp.where(c < MAINC, main, tail) * SLAB

    def gather(c, buf):
        for b in range(B):
            pltpu.async_copy(
                tab_hbm.at[idxs[b].at[pl.ds(c * ROWS, ROWS)]],
                rows_v.at[buf, pl.ds(b * ROWS, ROWS)],
                gsem.at[buf],
            )

    def gwait(c, buf):
        for b in range(B):
            pltpu.make_async_copy(
                tab_hbm.at[idxs[b].at[pl.ds(c * ROWS, ROWS)]],
                rows_v.at[buf, pl.ds(b * ROWS, ROWS)],
                gsem.at[buf],
            ).wait()

    def compute(c, buf):
        def do_v(v, _):
            coff = (v // 8) * 256 + (v % 8) * 16
            for p in range(CP):
                for b in range(B):
                    acc = None
                    for k in range(K):
                        r = rows_v[buf, b * ROWS + p * K + k, pl.ds(v * 16, 16)]
                        acc = r if acc is None else jnp.maximum(acc, r)
                    stage_v[buf, pl.ds(p * SLAB + b * 128 + coff, 16)] = acc
            return _

        lax.fori_loop(0, 16, do_v, 0, unroll=2)
        pltpu.async_copy(
            stage_v.at[buf],
            out_hbm.at[pl.ds(out_word(c), CP * SLAB)],
            osem.at[buf],
        )

    def owait(c, buf):
        pltpu.make_async_copy(
            stage_v.at[buf],
            out_hbm.at[pl.ds(out_word(c), CP * SLAB)],
            osem.at[buf],
        ).wait()

    ncr = MAINC + 1 + jnp.where(wid + NW < TAILCH, 1, 0)

    gather(0, 0)
    gather(1, 1)

    def step(c, _):
        buf = lax.rem(c, 3)

        @pl.when(c + 2 < ncr)
        def _prefetch():
            gather(c + 2, lax.rem(c + 2, 3))

        gwait(c, buf)

        @pl.when(c >= 3)
        def _drain_out():
            owait(c - 3, buf)

        compute(c, buf)
        return _

    lax.fori_loop(0, ncr, step, 0)
    owait(ncr - 3, lax.rem(ncr - 3, 3))
    owait(ncr - 2, lax.rem(ncr - 2, 3))
    owait(ncr - 1, lax.rem(ncr - 1, 3))


_sc_call = functools.partial(
    pl.kernel,
    out_type=jax.ShapeDtypeStruct((N_OUT * SLAB,), jnp.float32),
    mesh=plsc.VectorSubcoreMesh(core_axis_name="c", subcore_axis_name="s"),
    compiler_params=pltpu.CompilerParams(
        needs_layout_passes=False, use_tc_tiling_on_sc=True
    ),
    scratch_types=[
        pltpu.VMEM((K, MB * BLK), jnp.int32),
        pltpu.VMEM((PPT * K,), jnp.int32),
        pltpu.VMEM((PPT * K,), jnp.int32),
        pltpu.VMEM((3, B * ROWS, C), jnp.float32),
        pltpu.VMEM((3, CP * SLAB), jnp.float32),
        pltpu.SemaphoreType.DMA((3,)),
        pltpu.SemaphoreType.DMA((3,)),
    ],
)(_body)


def kernel(x, idx):
    tab = x.transpose(0, 2, 1).reshape(B * N_IN, C)
    idxt = idx.T
    tailg = idx[TAILBASE:].reshape(-1)
    out = _sc_call(tab, idxt, tailg)
    o4 = out.reshape(N_OUT, 2, B, 128)
    return o4.transpose(2, 1, 3, 0).reshape(B, C, N_OUT)
